# baseline (speedup 1.0000x reference)
import functools

import jax
import jax.numpy as jnp
from jax import lax
from jax.experimental import pallas as pl
from jax.experimental.pallas import tpu as pltpu
from jax.experimental.pallas import tpu_sc as plsc

VOCAB = 1000000
EMBED_DIM = 64
BATCH = 4096
SEQ = 200

NC = 2
NS = 16
NW = NC * NS

N_IDX = BATCH * SEQ
IDX_PER_W = N_IDX // NW
CHUNK = 128
ROWS_PER_W = IDX_PER_W // CHUNK
GROUP = 8
G_IDX = GROUP * CHUNK
N_GROUPS = ROWS_PER_W // GROUP


@functools.partial(
    pl.kernel,
    out_type=jax.ShapeDtypeStruct((N_IDX, EMBED_DIM), jnp.float32),
    mesh=plsc.VectorSubcoreMesh(core_axis_name="c", subcore_axis_name="s"),
    scratch_types=[
        pltpu.VMEM((GROUP, CHUNK), jnp.int32),
        pltpu.VMEM((G_IDX, EMBED_DIM), jnp.float32),
        pltpu.SemaphoreType.DMA,
    ],
    compiler_params=pltpu.CompilerParams(use_tc_tiling_on_sc=False),
)
def _gather_kernel(idx_hbm, table_hbm, out_hbm, idx_v, rows_v, sem):
    wid = lax.axis_index("s") * NC + lax.axis_index("c")
    row0 = wid * ROWS_PER_W
    base = wid * IDX_PER_W

    def body(g, carry):
        pltpu.sync_copy(idx_hbm.at[pl.ds(row0 + g * GROUP, GROUP)], idx_v)
        copies = []
        for j in range(GROUP):
            copies.append(
                pltpu.async_copy(
                    table_hbm.at[idx_v.at[j]],
                    rows_v.at[pl.ds(j * CHUNK, CHUNK)],
                    sem,
                )
            )
        for c in copies:
            c.wait()
        pltpu.sync_copy(rows_v, out_hbm.at[pl.ds(base + g * G_IDX, G_IDX)])
        return carry

    lax.fori_loop(0, N_GROUPS, body, 0)


def kernel(input_, weight):
    idx2d = input_.reshape(N_IDX // CHUNK, CHUNK)
    out = _gather_kernel(idx2d, weight)
    return out.reshape(BATCH, SEQ, EMBED_DIM)


# speedup vs baseline: 1.0136x; 1.0136x over previous
import functools

import jax
import jax.numpy as jnp
from jax import lax
from jax.experimental import pallas as pl
from jax.experimental.pallas import tpu as pltpu
from jax.experimental.pallas import tpu_sc as plsc

VOCAB = 1000000
EMBED_DIM = 64
BATCH = 4096
SEQ = 200

NC = 2
NS = 16
NW = NC * NS

N_IDX = BATCH * SEQ
IDX_PER_W = N_IDX // NW
CHUNK = 128
ROWS_PER_W = IDX_PER_W // CHUNK
GROUP_CHUNKS = 5
G_IDX = GROUP_CHUNKS * CHUNK
N_GROUPS = ROWS_PER_W // GROUP_CHUNKS
N_PAIRS = N_GROUPS // 2


@functools.partial(
    pl.kernel,
    out_type=jax.ShapeDtypeStruct((N_IDX, EMBED_DIM), jnp.float32),
    mesh=plsc.VectorSubcoreMesh(core_axis_name="c", subcore_axis_name="s"),
    scratch_types=[
        pltpu.VMEM((ROWS_PER_W, CHUNK), jnp.int32),
        pltpu.VMEM((G_IDX, EMBED_DIM), jnp.float32),
        pltpu.VMEM((G_IDX, EMBED_DIM), jnp.float32),
        pltpu.SemaphoreType.DMA,
        pltpu.SemaphoreType.DMA,
        pltpu.SemaphoreType.DMA,
        pltpu.SemaphoreType.DMA,
    ],
    compiler_params=pltpu.CompilerParams(use_tc_tiling_on_sc=False),
)
def _gather_kernel(idx_hbm, table_hbm, out_hbm, idx_v, rows0, rows1,
                   sg0, sg1, ss0, ss1):
    wid = lax.axis_index("s") * NC + lax.axis_index("c")
    base = wid * IDX_PER_W
    rows = (rows0, rows1)
    sg = (sg0, sg1)
    ss = (ss0, ss1)

    pltpu.sync_copy(idx_hbm.at[pl.ds(wid * ROWS_PER_W, ROWS_PER_W)], idx_v)

    def fire_gather(g, b):
        for j in range(GROUP_CHUNKS):
            pltpu.async_copy(
                table_hbm.at[idx_v.at[g * GROUP_CHUNKS + j]],
                rows[b].at[pl.ds(j * CHUNK, CHUNK)],
                sg[b],
            )

    def wait_gather(b):
        for j in range(GROUP_CHUNKS):
            pltpu.make_async_copy(
                table_hbm.at[idx_v.at[j]],
                rows[b].at[pl.ds(j * CHUNK, CHUNK)],
                sg[b],
            ).wait()

    def fire_store(g, b):
        pltpu.async_copy(rows[b], out_hbm.at[pl.ds(base + g * G_IDX, G_IDX)],
                         ss[b])

    def wait_store(b):
        pltpu.make_async_copy(rows[b], out_hbm.at[pl.ds(base, G_IDX)],
                              ss[b]).wait()

    fire_gather(0, 0)

    def body(i, carry):
        g0 = 2 * i
        g1 = g0 + 1
        @pl.when(i > 0)
        def _():
            wait_store(1)
        fire_gather(g1, 1)
        wait_gather(0)
        fire_store(g0, 0)
        @pl.when(i < N_PAIRS - 1)
        def _():
            wait_store(0)
            fire_gather(g0 + 2, 0)
        wait_gather(1)
        fire_store(g1, 1)
        return carry

    lax.fori_loop(0, N_PAIRS, body, 0)
    wait_store(0)
    wait_store(1)


def kernel(input_, weight):
    idx2d = input_.reshape(N_IDX // CHUNK, CHUNK)
    out = _gather_kernel(idx2d, weight)
    return out.reshape(BATCH, SEQ, EMBED_DIM)
